# baseline (speedup 1.0000x reference)
import functools

import jax
import jax.numpy as jnp
from jax.experimental import pallas as pl
from jax.experimental.pallas import tpu as pltpu

E = 8
K = 2
D = 768
EP = 128
T = 2048


def _moe_step(x_ref, wg_ref, we_ref, be_ref, o_ref, a0_s, a1_s, w0_s, w1_s):
    e = pl.program_id(0)

    @pl.when(e == 0)
    def _gating():
        logits = jnp.dot(x_ref[...], wg_ref[...],
                         preferred_element_type=jnp.float32)
        lane = jax.lax.broadcasted_iota(jnp.int32, logits.shape, 1)
        logits = jnp.where(lane < E, logits, -1e30)
        v0 = jnp.max(logits, axis=1, keepdims=True)
        a0 = jnp.min(jnp.where(logits == v0, lane, EP), axis=1,
                     keepdims=True)
        logits2 = jnp.where(lane == a0, -1e30, logits)
        v1 = jnp.max(logits2, axis=1, keepdims=True)
        a1 = jnp.min(jnp.where(logits2 == v1, lane, EP), axis=1,
                     keepdims=True)
        w0 = 1.0 / (1.0 + jnp.exp(v1 - v0))
        a0_s[...] = a0
        a1_s[...] = a1
        w0_s[...] = w0
        w1_s[...] = 1.0 - w0

    a0 = a0_s[...]
    a1 = a1_s[...]
    w_e = (jnp.where(a0 == e, w0_s[...], 0.0)
           + jnp.where(a1 == e, w1_s[...], 0.0))
    contrib = w_e * (jnp.dot(x_ref[...], we_ref[0],
                             preferred_element_type=jnp.float32)
                     + be_ref[0])

    @pl.when(e == 0)
    def _init():
        o_ref[...] = contrib

    @pl.when(e > 0)
    def _accum():
        o_ref[...] += contrib


@jax.jit
def _moe(xs, wg_pad, We, be):
    return pl.pallas_call(
        _moe_step,
        grid=(E,),
        in_specs=[
            pl.BlockSpec((T, D), lambda e: (0, 0)),
            pl.BlockSpec((D, EP), lambda e: (0, 0)),
            pl.BlockSpec((1, D, D), lambda e: (e, 0, 0)),
            pl.BlockSpec((1, 1, D), lambda e: (e, 0, 0)),
        ],
        out_specs=pl.BlockSpec((T, D), lambda e: (0, 0)),
        out_shape=jax.ShapeDtypeStruct((T, D), jnp.float32),
        scratch_shapes=[
            pltpu.VMEM((T, 1), jnp.int32),
            pltpu.VMEM((T, 1), jnp.int32),
            pltpu.VMEM((T, 1), jnp.float32),
            pltpu.VMEM((T, 1), jnp.float32),
        ],
    )(xs, wg_pad, We, be.reshape(E, 1, D))


def kernel(x, Wg, We, be):
    xs = x.reshape(-1, x.shape[-1])
    wg_pad = jnp.pad(Wg, ((0, 0), (0, EP - Wg.shape[1])))
    out = _moe(xs, wg_pad, We, be)
    return out.reshape(x.shape)
